# Initial kernel scaffold; baseline (speedup 1.0000x reference)
import jax
import jax.numpy as jnp
from jax.experimental import pallas as pl


def kernel(x, edge_index, W1, b1, W2, b2, W3, b3, W4, b4):
    raise NotImplementedError("write your pallas kernel here")


# speedup vs baseline: 6.9680x; 6.9680x over previous
import functools

import jax
import jax.numpy as jnp
from jax import lax
from jax.experimental import pallas as pl
from jax.experimental.pallas import tpu as pltpu
from jax.experimental.pallas import tpu_sc as plsc

NC = 2
NS = 16
NW = NC * NS
K = 128


def _deg_kernel_body(cpt, rpt, dstr_hbm, ones_hbm, zeros_hbm, out_hbm,
                     hist, dbuf, ones_v):
  c = lax.axis_index("c")
  s = lax.axis_index("s")
  wid = c * NS + s
  pltpu.sync_copy(ones_hbm, ones_v)
  pltpu.sync_copy(zeros_hbm, hist.at[pl.ds(s * rpt, rpt)])
  plsc.subcore_barrier()

  def step(t, carry):
    chunk = wid * cpt + t
    pltpu.sync_copy(dstr_hbm.at[chunk], dbuf.at[0])
    pltpu.sync_copy(ones_v, hist.at[dbuf.at[0]], add=True)
    return carry

  lax.fori_loop(0, cpt, step, 0)
  plsc.subcore_barrier()
  pltpu.sync_copy(hist.at[pl.ds(s * rpt, rpt)], out_hbm.at[c, s])


def _scatter_kernel_body(cpt, rpt, g_hbm, srcr_hbm, dstr_hbm, zeros_hbm,
                         out_hbm, agg, sbuf, dbuf, rows, sem):
  c = lax.axis_index("c")
  s = lax.axis_index("s")
  wid = c * NS + s
  pltpu.sync_copy(zeros_hbm, agg.at[pl.ds(s * rpt, rpt)])
  plsc.subcore_barrier()

  def step(t, carry):
    chunk = wid * cpt + t
    pltpu.sync_copy(srcr_hbm.at[chunk], sbuf.at[0])
    pltpu.sync_copy(dstr_hbm.at[chunk], dbuf.at[0])
    pltpu.async_copy(g_hbm.at[sbuf.at[0]], rows.at[0], sem).wait()
    pltpu.sync_copy(rows.at[0], agg.at[dbuf.at[0]], add=True)
    return carry

  lax.fori_loop(0, cpt, step, 0)
  plsc.subcore_barrier()
  pltpu.sync_copy(agg.at[pl.ds(s * rpt, rpt)], out_hbm.at[c, s])


def _make_sc_kernels(n, e):
  cpt = -(-e // (K * NW))
  npad = ((n + NS) + NS - 1) // NS * NS
  rpt = npad // NS
  mesh = plsc.VectorSubcoreMesh(core_axis_name="c", subcore_axis_name="s",
                                num_cores=NC, num_subcores=NS)

  deg_kernel = pl.kernel(
      functools.partial(_deg_kernel_body, cpt, rpt),
      out_type=jax.ShapeDtypeStruct((NC, NS, rpt, 128), jnp.float32),
      mesh=mesh,
      scratch_types=[
          pltpu.VMEM_SHARED((npad, 128), jnp.float32),
          pltpu.VMEM((1, K), jnp.int32),
          pltpu.VMEM((K, 128), jnp.float32),
      ],
  )

  scatter_kernel = pl.kernel(
      functools.partial(_scatter_kernel_body, cpt, rpt),
      out_type=jax.ShapeDtypeStruct((NC, NS, rpt, 128), jnp.float32),
      mesh=mesh,
      scratch_types=[
          pltpu.VMEM_SHARED((npad, 128), jnp.float32),
          pltpu.VMEM((1, K), jnp.int32),
          pltpu.VMEM((1, K), jnp.int32),
          pltpu.VMEM((1, K, 128), jnp.float32),
          pltpu.SemaphoreType.DMA,
      ],
  )
  return deg_kernel, scatter_kernel, cpt, npad, rpt


def _pro_body(degp_ref, x_ref, dis_ref, g_ref):
  deg = degp_ref[0, :, 0:1] + degp_ref[1, :, 0:1] + 1.0
  dis = lax.rsqrt(deg)
  dis_b = jnp.broadcast_to(dis, x_ref.shape)
  dis_ref[...] = dis_b
  g_ref[...] = x_ref[...] * dis_b


def _layer_body(relu, aggp_ref, h_ref, dis_ref, w_ref, b_ref, h_out, g_out):
  dis = dis_ref[...]
  h = h_ref[...]
  a = dis * (aggp_ref[0] + aggp_ref[1]) + h * dis * dis
  z = jnp.dot(a, w_ref[...], preferred_element_type=jnp.float32) + b_ref[...]
  if relu:
    z = jnp.maximum(z, 0.0)
  h_out[...] = z
  g_out[...] = z * dis


def _final_body(c_out, aggp_ref, h_ref, dis_ref, w_ref, b_ref, o_ref):
  dis = dis_ref[...]
  h = h_ref[...]
  a = dis * (aggp_ref[0] + aggp_ref[1]) + h * dis * dis
  z = jnp.dot(a, w_ref[...], preferred_element_type=jnp.float32) + b_ref[...]
  col = lax.broadcasted_iota(jnp.int32, z.shape, 1)
  zm = jnp.where(col < c_out, z, -1e30)
  m = jnp.max(zm, axis=1, keepdims=True)
  ssum = jnp.sum(jnp.exp(zm - m), axis=1, keepdims=True)
  o_ref[...] = z - m - jnp.log(ssum)


def _make_tc_kernels(n):
  blk = 1000
  grid = (n // blk,)
  f32 = jnp.float32

  row = lambda i: (i, 0)
  par = lambda i: (0, i, 0)
  fix = lambda i: (0, 0)

  pro = pl.pallas_call(
      _pro_body,
      grid=grid,
      in_specs=[pl.BlockSpec((2, blk, 128), par),
                pl.BlockSpec((blk, 128), row)],
      out_specs=[pl.BlockSpec((blk, 128), row),
                 pl.BlockSpec((blk, 128), row)],
      out_shape=[jax.ShapeDtypeStruct((n, 128), f32),
                 jax.ShapeDtypeStruct((n, 128), f32)],
  )

  def make_layer(relu):
    return pl.pallas_call(
        functools.partial(_layer_body, relu),
        grid=grid,
        in_specs=[pl.BlockSpec((2, blk, 128), par),
                  pl.BlockSpec((blk, 128), row),
                  pl.BlockSpec((blk, 128), row),
                  pl.BlockSpec((128, 128), fix),
                  pl.BlockSpec((1, 128), fix)],
        out_specs=[pl.BlockSpec((blk, 128), row),
                   pl.BlockSpec((blk, 128), row)],
        out_shape=[jax.ShapeDtypeStruct((n, 128), f32),
                   jax.ShapeDtypeStruct((n, 128), f32)],
    )

  def make_final(c_out):
    return pl.pallas_call(
        functools.partial(_final_body, c_out),
        grid=grid,
        in_specs=[pl.BlockSpec((2, blk, 128), par),
                  pl.BlockSpec((blk, 128), row),
                  pl.BlockSpec((blk, 128), row),
                  pl.BlockSpec((128, 128), fix),
                  pl.BlockSpec((1, 128), fix)],
        out_specs=pl.BlockSpec((blk, 128), row),
        out_shape=jax.ShapeDtypeStruct((n, 128), f32),
    )

  return pro, make_layer(True), make_final


@jax.jit
def kernel(x, edge_index, W1, b1, W2, b2, W3, b3, W4, b4):
  n, d = x.shape
  e = edge_index.shape[1]
  c_out = W4.shape[1]

  deg_kernel, scatter_kernel, cpt, npad, rpt = _make_sc_kernels(n, e)
  pro, layer_k, make_final = _make_tc_kernels(n)

  ep = NW * cpt * K
  src_p = jnp.concatenate(
      [edge_index[0], jnp.zeros((ep - e,), jnp.int32)]).reshape(NW * cpt, K)
  dst_p = jnp.concatenate(
      [edge_index[1], jnp.full((ep - e,), n, jnp.int32)]).reshape(NW * cpt, K)

  zeros128 = jnp.zeros((rpt, 128), jnp.float32)
  ones128 = jnp.ones((K, 128), jnp.float32)

  degp = deg_kernel(dst_p, ones128, zeros128)
  degp = degp.reshape(NC, npad, 128)[:, :n]
  dis_b, g = pro(degp, x)

  def agg(gv):
    p = scatter_kernel(gv, src_p, dst_p, zeros128)
    return p.reshape(NC, npad, 128)[:, :n]

  h = x
  for W, b in ((W1, b1), (W2, b2), (W3, b3)):
    h, g = layer_k(agg(g), h, dis_b, W, b.reshape(1, -1))

  w4p = jnp.zeros((d, 128), jnp.float32).at[:, :c_out].set(W4)
  b4p = jnp.zeros((1, 128), jnp.float32).at[0, :c_out].set(b4)
  out = make_final(c_out)(agg(g), h, dis_b, w4p, b4p)
  return out[:, :c_out]
